# baseline (speedup 1.0000x reference)
import jax
import jax.numpy as jnp
from jax import lax
from jax.experimental import pallas as pl
from jax.experimental.pallas import tpu as pltpu
from jax.experimental.pallas import tpu_sc as plsc

NC = 2
NS = 16
LANES = 16

B = 1024
L = 512
D = 128
BL = B * L
NW = NC * NS
K = 128
CHUNKS = BL // K // NW
CPS = L // K
NBUF = 4
NIDX = 8
ROUNDS = CHUNKS // (NBUF * 2)
GROUPS = K // LANES


def _comb_body(pos_ref, seg_ref, out_ref):
    out_ref[...] = seg_ref[...][:, None, :] + pos_ref[...][None, :, :]


def _build_comb(position_table, segment_table):
    comb = pl.pallas_call(
        _comb_body,
        in_specs=[
            pl.BlockSpec((L, D), lambda: (0, 0)),
            pl.BlockSpec((3, D), lambda: (0, 0)),
        ],
        out_specs=pl.BlockSpec((3, L, D), lambda: (0, 0, 0)),
        out_shape=jax.ShapeDtypeStruct((3, L, D), jnp.float32),
    )(position_table, segment_table)
    return comb.reshape(3 * L, D)


def _sc_body(seq_hbm, lab_hbm, tok_hbm, comb_hbm, out_hbm,
             comb_sp, idx_ring, lab_ring, cidx_ring, bufs,
             i_sem, c_sem, a_sem, w_sem):
    wid = lax.axis_index("s") * NC + lax.axis_index("c")
    w_base = wid * CHUNKS * K
    iota = lax.iota(jnp.int32, LANES)

    sid = lax.axis_index("s")
    rows = (3 * L) // NS
    pltpu.sync_copy(comb_hbm.at[pl.ds(sid * rows, rows)],
                    comb_sp.at[pl.ds(sid * rows, rows)])
    plsc.subcore_barrier()

    def issue_idx(k, slot):
        base = w_base + k * K
        pltpu.async_copy(seq_hbm.at[pl.ds(base, K)], idx_ring[slot],
                         i_sem[slot])
        pltpu.async_copy(lab_hbm.at[pl.ds(base, K)], lab_ring[slot],
                         i_sem[slot])

    def wait_idx(slot):
        for ring in (idx_ring, lab_ring):
            pltpu.make_async_copy(
                seq_hbm.at[pl.ds(0, K)], ring[slot], i_sem[slot]).wait()

    def prep_comb(bslot, islot):
        labs = lab_ring[islot]
        cidx = cidx_ring[islot]
        for g in range(GROUPS):
            sl = pl.ds(g * LANES, LANES)
            base = bslot * K + g * LANES
            cidx[sl] = labs[sl] * L + (iota + base)
        pltpu.async_copy(comb_sp.at[cidx], bufs[bslot], c_sem[bslot])

    def round_body(r, _):
        for u in range(NIDX):
            k = r * NIDX + u
            b = u % NBUF
            buf = bufs[b]
            nb_b = (u + 1) % NBUF
            nb_i = (u + 1) % NIDX
            pb_b = (u - 2) % NBUF
            pb_i = (u - 2) % NIDX

            pltpu.make_async_copy(
                comb_sp.at[cidx_ring[u]], buf, c_sem[b]).wait()
            pltpu.async_copy(tok_hbm.at[idx_ring[u]], buf, a_sem[b],
                             add=True)

            @pl.when(k + 2 < CHUNKS)
            def _prefetch_indices():
                issue_idx(k + 2, (u + 2) % NIDX)

            @pl.when(k + 1 < CHUNKS)
            def _stage_next():
                wait_idx(nb_i)

                @pl.when(k >= NBUF - 1)
                def _wait_writeout():
                    pltpu.make_async_copy(
                        bufs[nb_b], out_hbm.at[pl.ds(0, K)],
                        w_sem[nb_b]).wait()

                prep_comb(nb_b, nb_i)

            @pl.when(k >= 2)
            def _retire_prev():
                pltpu.make_async_copy(
                    tok_hbm.at[idx_ring[pb_i]], bufs[pb_b],
                    a_sem[pb_b]).wait()
                pltpu.async_copy(
                    bufs[pb_b], out_hbm.at[pl.ds(w_base + (k - 2) * K, K)],
                    w_sem[pb_b])
        return _

    issue_idx(0, 0)
    issue_idx(1, 1)
    wait_idx(0)
    prep_comb(0, 0)
    lax.fori_loop(0, ROUNDS, round_body, 0)
    for kk in (CHUNKS - 2, CHUNKS - 1):
        bb = kk % NBUF
        pltpu.make_async_copy(
            tok_hbm.at[idx_ring[kk % NIDX]], bufs[bb], a_sem[bb]).wait()
        pltpu.async_copy(bufs[bb], out_hbm.at[pl.ds(w_base + kk * K, K)],
                         w_sem[bb])
    for kk in range(CHUNKS - NBUF, CHUNKS):
        bb = kk % NBUF
        pltpu.make_async_copy(
            bufs[bb], out_hbm.at[pl.ds(0, K)], w_sem[bb]).wait()


def _sc_embed(seq_flat, lab_flat, token_table, comb):
    mesh = plsc.VectorSubcoreMesh(core_axis_name="c", subcore_axis_name="s")
    kfn = pl.kernel(
        _sc_body,
        out_type=jax.ShapeDtypeStruct((BL, D), jnp.float32),
        mesh=mesh,
        scratch_types=[
            pltpu.VMEM_SHARED((3 * L, D), jnp.float32),
            [pltpu.VMEM((K,), jnp.int32) for _ in range(NIDX)],
            [pltpu.VMEM((K,), jnp.int32) for _ in range(NIDX)],
            [pltpu.VMEM((K,), jnp.int32) for _ in range(NIDX)],
            [pltpu.VMEM((K, D), jnp.float32) for _ in range(NBUF)],
            [pltpu.SemaphoreType.DMA for _ in range(NIDX)],
            [pltpu.SemaphoreType.DMA for _ in range(NBUF)],
            [pltpu.SemaphoreType.DMA for _ in range(NBUF)],
            [pltpu.SemaphoreType.DMA for _ in range(NBUF)],
        ],
    )
    return kfn(seq_flat, lab_flat, token_table, comb)


@jax.jit
def _embed(seq_flat, lab_flat, token_table, position_table, segment_table):
    comb = _build_comb(position_table, segment_table)
    return _sc_embed(seq_flat, lab_flat, token_table, comb)


def kernel(sequence, segment_label, token_table, position_table, segment_table):
    seq_flat = sequence.reshape(BL).astype(jnp.int32)
    lab_flat = segment_label.reshape(BL).astype(jnp.int32)
    out = _embed(seq_flat, lab_flat, token_table, position_table,
                 segment_table)
    return out.reshape(B, L, D)


# speedup vs baseline: 1.0068x; 1.0068x over previous
import jax
import jax.numpy as jnp
from jax import lax
from jax.experimental import pallas as pl
from jax.experimental.pallas import tpu as pltpu
from jax.experimental.pallas import tpu_sc as plsc

NC = 2
NS = 16
LANES = 16

B = 1024
L = 512
D = 128
BL = B * L
NW = NC * NS
K = 128
CHUNKS = BL // K // NW
CPS = L // K
NBUF = 4
ROUNDS = CHUNKS // NBUF
GROUPS = K // LANES


def _comb_body(pos_ref, seg_ref, out_ref):
    out_ref[...] = seg_ref[...][:, None, :] + pos_ref[...][None, :, :]


def _build_comb(position_table, segment_table):
    comb = pl.pallas_call(
        _comb_body,
        in_specs=[
            pl.BlockSpec((L, D), lambda: (0, 0)),
            pl.BlockSpec((3, D), lambda: (0, 0)),
        ],
        out_specs=pl.BlockSpec((3, L, D), lambda: (0, 0, 0)),
        out_shape=jax.ShapeDtypeStruct((3, L, D), jnp.float32),
    )(position_table, segment_table)
    return comb.reshape(3 * L, D)


def _sc_body(seq_hbm, lab_hbm, tok_hbm, comb_hbm, out_hbm,
             comb_sp, idx_ring, lab_ring, cidx_ring, bufs,
             i_sem, c_sem, a_sem, w_sem):
    wid = lax.axis_index("s") * NC + lax.axis_index("c")
    w_base = wid * CHUNKS * K
    iota = lax.iota(jnp.int32, LANES)

    sid = lax.axis_index("s")
    rows = (3 * L) // NS
    pltpu.sync_copy(comb_hbm.at[pl.ds(sid * rows, rows)],
                    comb_sp.at[pl.ds(sid * rows, rows)])
    plsc.subcore_barrier()

    def issue_idx(k, slot):
        base = w_base + k * K
        pltpu.async_copy(seq_hbm.at[pl.ds(base, K)], idx_ring[slot],
                         i_sem[slot])
        pltpu.async_copy(lab_hbm.at[pl.ds(base, K)], lab_ring[slot],
                         i_sem[slot])

    def wait_idx(slot):
        for ring in (idx_ring, lab_ring):
            pltpu.make_async_copy(
                seq_hbm.at[pl.ds(0, K)], ring[slot], i_sem[slot]).wait()

    def prep_comb(k, slot):
        del k
        labs = lab_ring[slot]
        cidx = cidx_ring[slot]
        for g in range(GROUPS):
            sl = pl.ds(g * LANES, LANES)
            base = slot * K + g * LANES
            cidx[sl] = labs[sl] * L + (iota + base)
        pltpu.async_copy(comb_sp.at[cidx], bufs[slot], c_sem[slot])

    def round_body(r, _):
        for b in range(NBUF):
            k = r * NBUF + b
            buf = bufs[b]
            nb = (b + 1) % NBUF
            pb = (b - 1) % NBUF

            pltpu.make_async_copy(
                comb_sp.at[cidx_ring[b]], buf, c_sem[b]).wait()
            pltpu.async_copy(tok_hbm.at[idx_ring[b]], buf, a_sem[b],
                             add=True)

            @pl.when(k + 2 < CHUNKS)
            def _prefetch_indices():
                issue_idx(k + 2, (b + 2) % NBUF)

            @pl.when(k + 1 < CHUNKS)
            def _stage_next():
                wait_idx(nb)

                @pl.when(k >= NBUF - 1)
                def _wait_writeout():
                    pltpu.make_async_copy(
                        bufs[nb], out_hbm.at[pl.ds(0, K)], w_sem[nb]).wait()

                prep_comb(k + 1, nb)

            @pl.when(k >= 1)
            def _retire_prev():
                pltpu.make_async_copy(
                    tok_hbm.at[idx_ring[pb]], bufs[pb], a_sem[pb]).wait()
                pltpu.async_copy(
                    bufs[pb], out_hbm.at[pl.ds(w_base + (k - 1) * K, K)],
                    w_sem[pb])
        return _

    issue_idx(0, 0)
    issue_idx(1, 1)
    wait_idx(0)
    prep_comb(0, 0)
    lax.fori_loop(0, ROUNDS, round_body, 0)
    lb = (CHUNKS - 1) % NBUF
    pltpu.make_async_copy(
        tok_hbm.at[idx_ring[lb]], bufs[lb], a_sem[lb]).wait()
    pltpu.async_copy(bufs[lb],
                     out_hbm.at[pl.ds(w_base + (CHUNKS - 1) * K, K)],
                     w_sem[lb])
    for k in range(CHUNKS - NBUF, CHUNKS):
        b = k % NBUF
        pltpu.make_async_copy(
            bufs[b], out_hbm.at[pl.ds(0, K)], w_sem[b]).wait()


def _sc_embed(seq_flat, lab_flat, token_table, comb):
    mesh = plsc.VectorSubcoreMesh(core_axis_name="c", subcore_axis_name="s")
    kfn = pl.kernel(
        _sc_body,
        out_type=jax.ShapeDtypeStruct((BL, D), jnp.float32),
        mesh=mesh,
        scratch_types=[
            pltpu.VMEM_SHARED((3 * L, D), jnp.float32),
            [pltpu.VMEM((K,), jnp.int32) for _ in range(NBUF)],
            [pltpu.VMEM((K,), jnp.int32) for _ in range(NBUF)],
            [pltpu.VMEM((K,), jnp.int32) for _ in range(NBUF)],
            [pltpu.VMEM((K, D), jnp.float32) for _ in range(NBUF)],
            [pltpu.SemaphoreType.DMA for _ in range(NBUF)],
            [pltpu.SemaphoreType.DMA for _ in range(NBUF)],
            [pltpu.SemaphoreType.DMA for _ in range(NBUF)],
            [pltpu.SemaphoreType.DMA for _ in range(NBUF)],
        ],
    )
    return kfn(seq_flat, lab_flat, token_table, comb)


@jax.jit
def _embed(seq_flat, lab_flat, token_table, position_table, segment_table):
    comb = _build_comb(position_table, segment_table)
    return _sc_embed(seq_flat, lab_flat, token_table, comb)


def kernel(sequence, segment_label, token_table, position_table, segment_table):
    seq_flat = sequence.reshape(BL).astype(jnp.int32)
    lab_flat = segment_label.reshape(BL).astype(jnp.int32)
    out = _embed(seq_flat, lab_flat, token_table, position_table,
                 segment_table)
    return out.reshape(B, L, D)
